# Initial kernel scaffold; baseline (speedup 1.0000x reference)
import jax
import jax.numpy as jnp
from jax.experimental import pallas as pl


def kernel(matrix, hop_bias):
    raise NotImplementedError("write your pallas kernel here")


# speedup vs baseline: 6.8230x; 6.8230x over previous
import functools

import jax
import jax.numpy as jnp
from jax import lax
from jax.experimental import pallas as pl
from jax.experimental.pallas import tpu as pltpu
from jax.experimental.pallas import tpu_sc as plsc

HEADS = 32
TBL = 64
B = 8
N = 512
R = 8
CHUNKS = N // R
CHW = R * N
NW = 32
HPW = HEADS // (NW // B)


def _sc_body(mat_hbm, tbl_hbm, out_hbm, tbl_v, idx_v, cl_v, out_v):
    cid = lax.axis_index("c")
    sid = lax.axis_index("s")
    wid = sid * 2 + cid
    b = wid % B
    h0 = (wid // B) * HPW

    pltpu.sync_copy(tbl_hbm, tbl_v)

    def chunk_body(c, carry):
        moff = b * (N * N) + c * CHW
        pltpu.sync_copy(mat_hbm.at[pl.ds(moff, CHW)], idx_v)

        def clamp_body(i, carry2):
            m = idx_v[pl.ds(i * 16, 16)]
            clv = jnp.minimum(m, 51)
            clv = jnp.where(m < 0, 52, clv)
            cl_v[pl.ds(i * 16, 16)] = clv
            return carry2

        lax.fori_loop(0, CHW // 16, clamp_body, 0, unroll=4)

        def vec_body(i, carry2):
            iv = cl_v[pl.ds(i * 16, 16)]
            for h in range(HPW):
                hbase = (h0 + h) * TBL
                val = plsc.load_gather(tbl_v, [iv + hbase])
                out_v[pl.ds(h * CHW + i * 16, 16)] = val
            return carry2

        lax.fori_loop(0, CHW // 16, vec_body, 0, unroll=2)

        for h in range(HPW):
            ooff = (b * HEADS + h0 + h) * (N * N) + c * CHW
            pltpu.sync_copy(out_v.at[pl.ds(h * CHW, CHW)],
                            out_hbm.at[pl.ds(ooff, CHW)])
        return carry

    lax.fori_loop(0, CHUNKS, chunk_body, 0)


def kernel(matrix, hop_bias):
    mat = matrix.reshape(-1)
    tbl = jnp.zeros((HEADS, TBL), jnp.float32)
    tbl = tbl.at[:, :hop_bias.shape[0]].set(hop_bias.T).reshape(-1)

    mesh = plsc.VectorSubcoreMesh(core_axis_name="c", subcore_axis_name="s")
    run = functools.partial(
        pl.kernel,
        out_type=jax.ShapeDtypeStruct((B * HEADS * N * N,), jnp.float32),
        mesh=mesh,
        scratch_types=[
            pltpu.VMEM((HEADS * TBL,), jnp.float32),
            pltpu.VMEM((CHW,), jnp.int32),
            pltpu.VMEM((CHW,), jnp.int32),
            pltpu.VMEM((HPW * CHW,), jnp.float32),
        ],
        compiler_params=pltpu.CompilerParams(needs_layout_passes=False),
    )(_sc_body)

    out = run(mat, tbl)
    return out.reshape(B, HEADS, N, N)


# speedup vs baseline: 9.1555x; 1.3419x over previous
import functools

import jax
import jax.numpy as jnp
from jax import lax
from jax.experimental import pallas as pl
from jax.experimental.pallas import tpu as pltpu
from jax.experimental.pallas import tpu_sc as plsc

HEADS = 32
TBL = 64
B = 8
N = 512
R = 8
CHUNKS = N // R
CHW = R * N
NW = 32
HPW = HEADS // (NW // B)


def _sc_body(mat_hbm, tbl_hbm, out_hbm, tbl_v,
             idx0, idx1, out0, out1, sem_in, sem_out):
    cid = lax.axis_index("c")
    sid = lax.axis_index("s")
    wid = sid * 2 + cid
    b = wid % B
    h0 = (wid // B) * HPW
    plane0 = b * HEADS + h0
    mbase = b * (N * N)
    hbase = h0 * TBL

    idx_bufs = [idx0, idx1]
    out_bufs = [out0, out1]

    pltpu.sync_copy(tbl_hbm, tbl_v)

    def in_dma(c, buf):
        return pltpu.make_async_copy(
            mat_hbm.at[pl.ds(mbase + c * CHW, CHW)], buf, sem_in)

    def out_dma(c, buf):
        return pltpu.make_async_copy(
            buf, out_hbm.at[pl.ds(plane0, HPW), pl.ds(c * CHW, CHW)], sem_out)

    in_dma(0, idx_bufs[0]).start()

    def pair_body(t, carry):
        for p in range(2):
            c = 2 * t + p
            ibuf = idx_bufs[p]
            obuf = out_bufs[p]

            in_dma(c, ibuf).wait()

            @pl.when(c + 1 < CHUNKS)
            def _():
                in_dma(c + 1, idx_bufs[1 - p]).start()

            @pl.when(c >= 2)
            def _():
                out_dma(c - 2, obuf).wait()

            def vec_body(i, carry2):
                m = ibuf[pl.ds(i * 16, 16)]
                iv = jnp.where(m < 0, 52, jnp.minimum(m, 51)) + hbase
                for h in range(HPW):
                    val = plsc.load_gather(tbl_v, [iv + h * TBL])
                    obuf[h, pl.ds(i * 16, 16)] = val
                return carry2

            lax.fori_loop(0, CHW // 16, vec_body, 0, unroll=2)

            out_dma(c, obuf).start()
        return carry

    lax.fori_loop(0, CHUNKS // 2, pair_body, 0)

    out_dma(CHUNKS - 2, out_bufs[0]).wait()
    out_dma(CHUNKS - 1, out_bufs[1]).wait()


def kernel(matrix, hop_bias):
    mat = matrix.reshape(-1)
    tbl = jnp.zeros((HEADS, TBL), jnp.float32)
    tbl = tbl.at[:, :hop_bias.shape[0]].set(hop_bias.T).reshape(-1)

    mesh = plsc.VectorSubcoreMesh(core_axis_name="c", subcore_axis_name="s")
    run = functools.partial(
        pl.kernel,
        out_type=jax.ShapeDtypeStruct((B * HEADS, N * N), jnp.float32),
        mesh=mesh,
        scratch_types=[
            pltpu.VMEM((HEADS * TBL,), jnp.float32),
            pltpu.VMEM((CHW,), jnp.int32),
            pltpu.VMEM((CHW,), jnp.int32),
            pltpu.VMEM((HPW, CHW), jnp.float32),
            pltpu.VMEM((HPW, CHW), jnp.float32),
            pltpu.SemaphoreType.DMA,
            pltpu.SemaphoreType.DMA,
        ],
        compiler_params=pltpu.CompilerParams(needs_layout_passes=False),
    )(_sc_body)

    out = run(mat, tbl)
    return out.reshape(B, HEADS, N, N)
